# baseline (speedup 1.0000x reference)
import functools

import jax
import jax.numpy as jnp
from jax import lax
from jax.experimental import pallas as pl
from jax.experimental.pallas import tpu as pltpu
from jax.experimental.pallas import tpu_sc as plsc

B = 16384
F = 26
V = 100000
D = 16
DENSE = 13

NC = 2
NS = 16
NW = NC * NS
SPW = B // NW
IPW = SPW * F
IDX_ROWS = IPW // 128
C = 64
NCHUNK = SPW // C
DPC = C * F // 128


def _body(x_hbm, xd_hbm, emb_hbm, lin_hbm, w_hbm, out_hbm,
          xi_v, rows_v, lin_v, xd_v, t_v, w_v, out_v, sem_e, sem_l):
    wid = lax.axis_index("s") * NC + lax.axis_index("c")
    pltpu.sync_copy(x_hbm.at[wid], xi_v)
    pltpu.sync_copy(xd_hbm.at[wid], xd_v)
    pltpu.sync_copy(w_hbm, w_v)

    iota16 = lax.iota(jnp.int32, 16)
    wreg = w_v[pl.ds(0, 16)]

    def trans_body(jj, _):
        p0 = jj * 128
        for k in range(8):
            pos = p0 + (k * 16) + iota16
            f = pos % F
            xi_v[jj, pl.ds(k * 16, 16)] = xi_v[jj, pl.ds(k * 16, 16)] + f * V
        return 0
    lax.fori_loop(0, IDX_ROWS, trans_body, 0, unroll=False)

    for g in range(NCHUNK):
        copies = []
        for j in range(DPC):
            r = g * DPC + j
            cp = pltpu.make_async_copy(
                emb_hbm.at[xi_v.at[r]], rows_v.at[pl.ds(j * 128, 128), :],
                sem_e)
            cp.start()
            copies.append(cp)
            cpl = pltpu.make_async_copy(
                lin_hbm.at[xi_v.at[r]], lin_v.at[pl.ds(r * 128, 128)],
                sem_l)
            cpl.start()
            copies.append(cpl)
        for cp in copies:
            cp.wait()

        def fm_body(i, _):
            base = i * F
            s0 = rows_v[base, :]
            ss0 = s0 * s0
            s1 = rows_v[base + 1, :]
            ss1 = s1 * s1
            for f in range(2, F, 2):
                e0 = rows_v[base + f, :]
                s0 = s0 + e0
                ss0 = ss0 + e0 * e0
                e1 = rows_v[base + f + 1, :]
                s1 = s1 + e1
                ss1 = ss1 + e1 * e1
            s = s0 + s1
            t_v[pl.ds(i * 16, 16)] = s * s - (ss0 + ss1)
            return 0
        lax.fori_loop(0, C, fm_body, 0, unroll=False)

        def grp_body(gr, _):
            sb = g * C + gr * 16
            facc = plsc.load_gather(t_v, [iota16 * 16 + gr * 256])
            for dd in range(1, D):
                facc = facc + plsc.load_gather(
                    t_v, [iota16 * 16 + (gr * 256 + dd)])
            lacc = plsc.load_gather(lin_v, [iota16 * F + sb * F])
            for f in range(1, F):
                lacc = lacc + plsc.load_gather(
                    lin_v, [iota16 * F + (sb * F + f)])
            dacc = wreg[0] * plsc.load_gather(xd_v, [iota16 * DENSE + sb * DENSE])
            for jj in range(1, DENSE):
                dacc = dacc + wreg[jj] * plsc.load_gather(
                    xd_v, [iota16 * DENSE + (sb * DENSE + jj)])
            out_v[pl.ds(sb, 16)] = lacc + dacc + 0.5 * facc + wreg[DENSE]
            return 0
        lax.fori_loop(0, C // 16, grp_body, 0, unroll=False)

    pltpu.sync_copy(out_v, out_hbm.at[pl.ds(wid * SPW, SPW)])


@jax.jit
def _fm(x, xd, emb, lin, w):
    mesh = plsc.VectorSubcoreMesh(
        core_axis_name="c", subcore_axis_name="s",
        num_cores=NC, num_subcores=NS)
    return pl.kernel(
        _body,
        out_type=jax.ShapeDtypeStruct((B,), jnp.float32),
        mesh=mesh,
        scratch_types=[
            pltpu.VMEM((IDX_ROWS, 128), jnp.int32),
            pltpu.VMEM((C * F, D), jnp.float32),
            pltpu.VMEM((IPW,), jnp.float32),
            pltpu.VMEM((SPW * DENSE,), jnp.float32),
            pltpu.VMEM((C * D,), jnp.float32),
            pltpu.VMEM((16,), jnp.float32),
            pltpu.VMEM((SPW,), jnp.float32),
            pltpu.SemaphoreType.DMA,
            pltpu.SemaphoreType.DMA,
        ],
        compiler_params=pltpu.CompilerParams(
            needs_layout_passes=False, use_tc_tiling_on_sc=False),
    )(x, xd, emb, lin, w)


def kernel(x_sparse, x_dense, emb_tables, lin_tables, lin_dense_w, bias):
    x = x_sparse.astype(jnp.int32).reshape(NW, IDX_ROWS, 128)
    xd = x_dense.reshape(NW, SPW * DENSE)
    emb = emb_tables.reshape(F * V, D)
    lin = lin_tables.reshape(F * V)
    w = jnp.concatenate([lin_dense_w, bias,
                         jnp.zeros((2,), jnp.float32)])
    return _fm(x, xd, emb, lin, w)
